# baseline (speedup 1.0000x reference)
import functools

import jax
import jax.numpy as jnp
from jax import lax
from jax.experimental import pallas as pl
from jax.experimental.pallas import tpu as pltpu
from jax.experimental.pallas import tpu_sc as plsc

_NUM_CORES = 2
_NUM_SUBCORES = 16
_NW = _NUM_CORES * _NUM_SUBCORES
_CHUNK = 128
_NBUF = 5
_K = 3
_SC_UNIT = _NW * _CHUNK * _NBUF
_SC_FRAC = 0.9
_R_TC = 128
_TC_UNIT = 2 * _R_TC


@functools.lru_cache(maxsize=None)
def _build_sc(n_rows: int, d: int, out_rows: int):
    assert n_rows % _SC_UNIT == 0 and out_rows >= n_rows
    rows_per_w = n_rows // _NW
    n_chunks = rows_per_w // _CHUNK
    n_rounds = n_chunks // _NBUF
    assert n_rounds >= 2

    mesh = plsc.VectorSubcoreMesh(core_axis_name="c", subcore_axis_name="s")

    @functools.partial(
        pl.kernel,
        mesh=mesh,
        out_type=jax.ShapeDtypeStruct((out_rows, d), jnp.float32),
        scratch_types=[
            pltpu.VMEM((rows_per_w,), jnp.int32),
            pltpu.VMEM((_NBUF, _CHUNK, d), jnp.float32),
            pltpu.SemaphoreType.DMA((_NBUF,)),
            pltpu.SemaphoreType.DMA((_NBUF,)),
        ],
    )
    def sc_gather(idx_hbm, table_hbm, out_hbm, idx_all, rows, gsem, osem):
        wid = lax.axis_index("s") * _NUM_CORES + lax.axis_index("c")
        base = wid * rows_per_w
        pltpu.sync_copy(idx_hbm.at[pl.ds(base, rows_per_w)], idx_all)

        def gather_copy(g, b):
            return pltpu.make_async_copy(
                table_hbm.at[idx_all.at[pl.ds(g * _CHUNK, _CHUNK)]],
                rows.at[b],
                gsem.at[b],
            )

        def out_copy(g, b):
            return pltpu.make_async_copy(
                rows.at[b],
                out_hbm.at[pl.ds(base + g * _CHUNK, _CHUNK)],
                osem.at[b],
            )

        def step(g, j, drain_out, issue_gather):
            gather_copy(g, j).wait()
            out_copy(g, j).start()
            if issue_gather:
                b2 = (j + _K) % _NBUF
                if drain_out:
                    out_copy(g, b2).wait()
                gather_copy(g + _K, b2).start()

        for b in range(_K):
            gather_copy(b, b).start()

        for j in range(_NBUF):
            step(j, j, drain_out=(j + _K >= _NBUF), issue_gather=True)

        def round_body(i, carry):
            g0 = i * _NBUF
            for j in range(_NBUF):
                step(g0 + j, j, drain_out=True, issue_gather=True)
            return carry

        lax.fori_loop(1, n_rounds - 1, round_body, 0)

        g0 = (n_rounds - 1) * _NBUF
        for j in range(_NBUF):
            g = g0 + j
            step(g, j, drain_out=True, issue_gather=(g + _K < n_chunks))

        for j in range(_NBUF):
            out_copy(g0 + j, j).wait()

    return sc_gather


@functools.lru_cache(maxsize=None)
def _build_tc(m: int, d: int):
    assert m % _TC_UNIT == 0
    n_ch = m // _R_TC
    n_pairs = n_ch // 2

    def tc_gather(idx_hbm, table_hbm, out_hbm, idx_sm, buf, isem, gsem, osem):
        def idx_cp(c, b):
            return pltpu.make_async_copy(
                idx_hbm.at[pl.ds(c * _R_TC, _R_TC)], idx_sm.at[b], isem.at[b]
            )

        def out_cp(c, b):
            return pltpu.make_async_copy(
                buf.at[b], out_hbm.at[pl.ds(c * _R_TC, _R_TC)], osem.at[b]
            )

        def chunk(c, b, drain_out, prefetch_idx):
            idx_cp(c, b).wait()
            if drain_out:
                out_cp(c - 2, b).wait()
            for j in range(_R_TC):
                row = idx_sm[b, j]
                pltpu.make_async_copy(
                    table_hbm.at[pl.ds(row, 1)],
                    buf.at[b, pl.ds(j, 1)],
                    gsem.at[b],
                ).start()
            if prefetch_idx:
                idx_cp(c + 2, b).start()
            pltpu.make_async_copy(
                table_hbm.at[pl.ds(0, _R_TC)], buf.at[b], gsem.at[b]
            ).wait()
            out_cp(c, b).start()

        idx_cp(0, 0).start()
        idx_cp(1, 1).start()

        chunk(0, 0, drain_out=False, prefetch_idx=n_ch > 2)
        chunk(1, 1, drain_out=False, prefetch_idx=n_ch > 3)

        def pair_body(i, carry):
            c0 = 2 * i
            chunk(c0, 0, drain_out=True, prefetch_idx=True)
            chunk(c0 + 1, 1, drain_out=True, prefetch_idx=True)
            return carry

        lax.fori_loop(1, n_pairs - 1, pair_body, 0)

        if n_pairs > 1:
            c0 = 2 * (n_pairs - 1)
            chunk(c0, 0, drain_out=True, prefetch_idx=False)
            chunk(c0 + 1, 1, drain_out=True, prefetch_idx=False)

        out_cp(n_ch - 2, 0).wait()
        out_cp(n_ch - 1, 1).wait()

    return pl.pallas_call(
        tc_gather,
        out_shape=jax.ShapeDtypeStruct((m, d), jnp.float32),
        in_specs=[
            pl.BlockSpec(memory_space=pl.ANY),
            pl.BlockSpec(memory_space=pl.ANY),
        ],
        out_specs=pl.BlockSpec(memory_space=pl.ANY),
        scratch_shapes=[
            pltpu.SMEM((2, _R_TC), jnp.int32),
            pltpu.VMEM((2, _R_TC, d), jnp.float32),
            pltpu.SemaphoreType.DMA((2,)),
            pltpu.SemaphoreType.DMA((2,)),
            pltpu.SemaphoreType.DMA((2,)),
        ],
    )


@functools.lru_cache(maxsize=None)
def _build_merge(n: int, m: int, d: int):
    def merge(full_in, tc_in, out_ref, sem):
        pltpu.make_async_copy(tc_in, out_ref.at[pl.ds(n - m, m)], sem).start()
        pltpu.make_async_copy(tc_in, out_ref.at[pl.ds(n - m, m)], sem).wait()

    return pl.pallas_call(
        merge,
        out_shape=jax.ShapeDtypeStruct((n, d), jnp.float32),
        in_specs=[
            pl.BlockSpec(memory_space=pl.ANY),
            pl.BlockSpec(memory_space=pl.ANY),
        ],
        out_specs=pl.BlockSpec(memory_space=pl.ANY),
        scratch_shapes=[pltpu.SemaphoreType.DMA],
        input_output_aliases={0: 0},
    )


def kernel(token_seq_inputs, embedding_table):
    batch, seq = token_seq_inputs.shape
    _, d = embedding_table.shape
    idx = token_seq_inputs.reshape(-1).astype(jnp.int32)
    n = idx.shape[0]

    sc_rows = int(round(n * _SC_FRAC / _SC_UNIT)) * _SC_UNIT
    sc_rows = max(_SC_UNIT, min(sc_rows, (n // _SC_UNIT) * _SC_UNIT))
    tc_rows = n - sc_rows
    if tc_rows == 0:
        out = _build_sc(sc_rows, d, sc_rows)(idx, embedding_table)
        return out.reshape(batch, seq, d)

    out_sc = _build_sc(sc_rows, d, n)(idx, embedding_table)

    tc_idx = idx[sc_rows:]
    tc_pad = (-tc_rows) % _TC_UNIT
    if tc_pad:
        tc_idx = jnp.concatenate([tc_idx, jnp.zeros((tc_pad,), jnp.int32)])
    out_tc = _build_tc(tc_rows + tc_pad, d)(tc_idx, embedding_table)
    if tc_pad:
        out_tc = out_tc[:tc_rows]

    out = _build_merge(n, tc_rows, d)(out_sc, out_tc)
    return out.reshape(batch, seq, d)


# speedup vs baseline: 2.6255x; 2.6255x over previous
import functools

import jax
import jax.numpy as jnp
from jax import lax
from jax.experimental import pallas as pl
from jax.experimental.pallas import tpu as pltpu
from jax.experimental.pallas import tpu_sc as plsc

_NUM_CORES = 2
_NUM_SUBCORES = 16
_NW = _NUM_CORES * _NUM_SUBCORES
_CHUNK = 64
_NBUF = 10
_K = 8


@functools.lru_cache(maxsize=None)
def _build(n_rows: int, d: int):
    assert n_rows % (_NW * _CHUNK * _NBUF) == 0
    rows_per_w = n_rows // _NW
    n_chunks = rows_per_w // _CHUNK
    n_rounds = n_chunks // _NBUF
    assert n_rounds >= 2

    mesh = plsc.VectorSubcoreMesh(core_axis_name="c", subcore_axis_name="s")

    @functools.partial(
        pl.kernel,
        mesh=mesh,
        out_type=jax.ShapeDtypeStruct((n_rows, d), jnp.float32),
        scratch_types=[
            pltpu.VMEM((rows_per_w,), jnp.int32),
            pltpu.VMEM((_NBUF, _CHUNK, d), jnp.float32),
            pltpu.SemaphoreType.DMA((_NBUF,)),
            pltpu.SemaphoreType.DMA((_NBUF,)),
        ],
    )
    def gather_kernel(idx_hbm, table_hbm, out_hbm, idx_all, rows, gsem, osem):
        wid = lax.axis_index("s") * _NUM_CORES + lax.axis_index("c")
        base = wid * rows_per_w
        pltpu.sync_copy(idx_hbm.at[pl.ds(base, rows_per_w)], idx_all)

        def gather_copy(g, b):
            return pltpu.make_async_copy(
                table_hbm.at[idx_all.at[pl.ds(g * _CHUNK, _CHUNK)]],
                rows.at[b],
                gsem.at[b],
            )

        def out_copy(g, b):
            return pltpu.make_async_copy(
                rows.at[b],
                out_hbm.at[pl.ds(base + g * _CHUNK, _CHUNK)],
                osem.at[b],
            )

        def step(g, j, drain_out, issue_gather):
            gather_copy(g, j).wait()
            out_copy(g, j).start()
            if issue_gather:
                b2 = (j + _K) % _NBUF
                if drain_out:
                    out_copy(g, b2).wait()
                gather_copy(g + _K, b2).start()

        for b in range(_K):
            gather_copy(b, b).start()

        for j in range(_NBUF):
            step(j, j, drain_out=(j + _K >= _NBUF), issue_gather=True)

        def round_body(i, carry):
            g0 = i * _NBUF
            for j in range(_NBUF):
                step(g0 + j, j, drain_out=True, issue_gather=True)
            return carry

        lax.fori_loop(1, n_rounds - 1, round_body, 0)

        g0 = (n_rounds - 1) * _NBUF
        for j in range(_NBUF):
            g = g0 + j
            step(g, j, drain_out=True, issue_gather=(g + _K < n_chunks))

        for j in range(_NBUF):
            out_copy(g0 + j, j).wait()

    return gather_kernel


def kernel(token_seq_inputs, embedding_table):
    batch, seq = token_seq_inputs.shape
    _, d = embedding_table.shape
    idx = token_seq_inputs.reshape(-1).astype(jnp.int32)
    n = idx.shape[0]
    pad = (-n) % (_NW * _CHUNK * _NBUF)
    if pad:
        idx = jnp.concatenate([idx, jnp.zeros((pad,), jnp.int32)])
    out = _build(n + pad, d)(idx, embedding_table)
    if pad:
        out = out[:n]
    return out.reshape(batch, seq, d)
